# baseline (speedup 1.0000x reference)
import functools

import jax
import jax.numpy as jnp
from jax import lax
from jax.experimental import pallas as pl
from jax.experimental.pallas import tpu as pltpu
from jax.experimental.pallas import tpu_sc as plsc

NC = 2
NS = 16
NW = NC * NS


@functools.lru_cache(maxsize=None)
def _make(B, D, n_chunks, chunk):
    mesh = plsc.VectorSubcoreMesh(
        core_axis_name="c", subcore_axis_name="s",
        num_cores=NC, num_subcores=NS)
    b_per_w = n_chunks * chunk

    @functools.partial(
        pl.kernel,
        out_type=jax.ShapeDtypeStruct((B, D), jnp.float32),
        mesh=mesh,
        scratch_types=[
            pltpu.VMEM((n_chunks, chunk), jnp.int32),
            pltpu.VMEM((chunk, D), jnp.float32),
            pltpu.SemaphoreType.DMA,
        ],
        compiler_params=pltpu.CompilerParams(use_tc_tiling_on_sc=False),
    )
    def k(idx_hbm, table_hbm, out_hbm, idx_v, rows_v, sem):
        wid = lax.axis_index("s") * NC + lax.axis_index("c")
        base = wid * b_per_w
        pltpu.sync_copy(idx_hbm.at[wid], idx_v)

        def body(j, carry):
            pltpu.async_copy(table_hbm.at[idx_v.at[j]], rows_v, sem).wait()
            pltpu.sync_copy(rows_v, out_hbm.at[pl.ds(base + j * chunk, chunk)])
            return carry

        lax.fori_loop(0, n_chunks, body, 0)

    return k


def kernel(inputs, table):
    S0, S1 = inputs.shape
    B = S0 * S1
    D = table.shape[1]
    chunk = 128
    n_chunks = B // (NW * chunk)
    idx = inputs.reshape(NW, n_chunks, chunk).astype(jnp.int32)
    out = _make(B, D, n_chunks, chunk)(idx, table)
    return out.reshape(S0, S1, D)


# speedup vs baseline: 1.1148x; 1.1148x over previous
import functools

import jax
import jax.numpy as jnp
from jax import lax
from jax.experimental import pallas as pl
from jax.experimental.pallas import tpu as pltpu
from jax.experimental.pallas import tpu_sc as plsc

NC = 2
NS = 16
NW = NC * NS

NBUF = 8
K = 4


@functools.lru_cache(maxsize=None)
def _make(B, D, n_chunks, chunk):
    mesh = plsc.VectorSubcoreMesh(
        core_axis_name="c", subcore_axis_name="s",
        num_cores=NC, num_subcores=NS)
    b_per_w = n_chunks * chunk
    assert n_chunks % NBUF == 0

    @functools.partial(
        pl.kernel,
        out_type=jax.ShapeDtypeStruct((B, D), jnp.float32),
        mesh=mesh,
        scratch_types=[
            pltpu.VMEM((n_chunks, chunk), jnp.int32),
            pltpu.VMEM((NBUF, chunk, D), jnp.float32),
            pltpu.SemaphoreType.DMA((NBUF,)),
            pltpu.SemaphoreType.DMA((NBUF,)),
        ],
        compiler_params=pltpu.CompilerParams(use_tc_tiling_on_sc=False),
    )
    def k(idx_hbm, table_hbm, out_hbm, idx_v, bufs, gsem, osem):
        wid = lax.axis_index("s") * NC + lax.axis_index("c")
        base = wid * b_per_w
        pltpu.sync_copy(idx_hbm.at[wid], idx_v)

        def gather(c, b):
            return pltpu.make_async_copy(
                table_hbm.at[idx_v.at[c]], bufs.at[b], gsem.at[b])

        def write(c, b):
            return pltpu.make_async_copy(
                bufs.at[b], out_hbm.at[pl.ds(base + c * chunk, chunk)],
                osem.at[b])

        for c in range(K):
            gather(c, c % NBUF).start()

        def step(g, carry):
            for b in range(NBUF):
                c = g * NBUF + b
                a = c + K
                ba = (b + K) % NBUF
                w = a - NBUF

                @pl.when(a < n_chunks)
                def _():
                    @pl.when(w >= 0)
                    def _():
                        write(w, ba).wait()
                    gather(a, ba).start()

                gather(c, b).wait()
                write(c, b).start()
            return carry

        lax.fori_loop(0, n_chunks // NBUF, step, 0)

        for c in range(n_chunks - NBUF, n_chunks):
            write(c, c % NBUF).wait()

    return k


def kernel(inputs, table):
    S0, S1 = inputs.shape
    B = S0 * S1
    D = table.shape[1]
    chunk = 128
    n_chunks = B // (NW * chunk)
    idx = inputs.reshape(NW, n_chunks, chunk).astype(jnp.int32)
    out = _make(B, D, n_chunks, chunk)(idx, table)
    return out.reshape(S0, S1, D)


# speedup vs baseline: 1.3622x; 1.2219x over previous
import functools

import jax
import jax.numpy as jnp
from jax import lax
from jax.experimental import pallas as pl
from jax.experimental.pallas import tpu as pltpu
from jax.experimental.pallas import tpu_sc as plsc

NC = 2
NS = 16
NW = NC * NS

NBUF = 4
K = 2


@functools.lru_cache(maxsize=None)
def _make(B, W, n_chunks, chunk):
    mesh = plsc.VectorSubcoreMesh(
        core_axis_name="c", subcore_axis_name="s",
        num_cores=NC, num_subcores=NS)
    b_per_w = n_chunks * chunk
    assert n_chunks % NBUF == 0

    @functools.partial(
        pl.kernel,
        out_type=jax.ShapeDtypeStruct((B, W), jnp.float32),
        mesh=mesh,
        scratch_types=[
            pltpu.VMEM((n_chunks, chunk), jnp.int32),
            pltpu.VMEM((NBUF, chunk, W), jnp.float32),
            pltpu.SemaphoreType.DMA((NBUF,)),
            pltpu.SemaphoreType.DMA((NBUF,)),
        ],
        compiler_params=pltpu.CompilerParams(use_tc_tiling_on_sc=False),
    )
    def k(idx_hbm, table_hbm, out_hbm, idx_v, bufs, gsem, osem):
        wid = lax.axis_index("s") * NC + lax.axis_index("c")
        base = wid * b_per_w
        pltpu.sync_copy(idx_hbm.at[wid], idx_v)

        def gather(c, b):
            return pltpu.make_async_copy(
                table_hbm.at[idx_v.at[c]], bufs.at[b], gsem.at[b])

        def write(c, b):
            return pltpu.make_async_copy(
                bufs.at[b], out_hbm.at[pl.ds(base + c * chunk, chunk)],
                osem.at[b])

        for c in range(K):
            gather(c, c % NBUF).start()

        def step(g, carry):
            for b in range(NBUF):
                c = g * NBUF + b
                a = c + K
                ba = (b + K) % NBUF
                w = a - NBUF

                @pl.when(a < n_chunks)
                def _():
                    @pl.when(w >= 0)
                    def _():
                        write(w, ba).wait()
                    gather(a, ba).start()

                gather(c, b).wait()
                write(c, b).start()
            return carry

        lax.fori_loop(0, n_chunks // NBUF, step, 0)

        for c in range(n_chunks - NBUF, n_chunks):
            write(c, c % NBUF).wait()

    return k


def kernel(inputs, table):
    S0, S1 = inputs.shape
    B = S0 * S1
    V, D = table.shape
    W = 128
    chunk = 128
    n_chunks = B // (NW * chunk)
    idx = inputs.reshape(NW, n_chunks, chunk).astype(jnp.int32)
    table_p = jnp.pad(table, ((0, 0), (0, W - D)))
    out_p = _make(B, W, n_chunks, chunk)(idx, table_p)
    return out_p.reshape(S0, S1, W)[:, :, :D]


# speedup vs baseline: 1.4722x; 1.0808x over previous
import functools

import jax
import jax.numpy as jnp
from jax import lax
from jax.experimental import pallas as pl
from jax.experimental.pallas import tpu as pltpu
from jax.experimental.pallas import tpu_sc as plsc

NC = 2
NS = 16
NW = NC * NS

NBUF = 4
K = 2


@functools.lru_cache(maxsize=None)
def _make(B, W, n_chunks, chunk):
    mesh = plsc.VectorSubcoreMesh(
        core_axis_name="c", subcore_axis_name="s",
        num_cores=NC, num_subcores=NS)
    b_per_w = n_chunks * chunk
    assert n_chunks % NBUF == 0

    @functools.partial(
        pl.kernel,
        out_type=jax.ShapeDtypeStruct((B, W), jnp.float32),
        mesh=mesh,
        scratch_types=[
            pltpu.VMEM((n_chunks, chunk), jnp.int32),
            pltpu.VMEM((NBUF, chunk, W), jnp.float32),
            pltpu.SemaphoreType.DMA((NBUF,)),
            pltpu.SemaphoreType.DMA((NBUF,)),
        ],
        compiler_params=pltpu.CompilerParams(use_tc_tiling_on_sc=False),
    )
    def k(idx_hbm, table_hbm, out_hbm, idx_v, bufs, gsem, osem):
        wid = lax.axis_index("s") * NC + lax.axis_index("c")
        base = wid * b_per_w
        pltpu.sync_copy(idx_hbm.at[wid], idx_v)

        def gather(c, b):
            return pltpu.make_async_copy(
                table_hbm.at[idx_v.at[c]], bufs.at[b], gsem.at[b])

        def write(c, b):
            return pltpu.make_async_copy(
                bufs.at[b, :, pl.ds(0, 64)],
                out_hbm.at[pl.ds(base + c * chunk, chunk), pl.ds(0, 64)],
                osem.at[b])

        for c in range(K):
            gather(c, c % NBUF).start()

        def step(g, carry):
            for b in range(NBUF):
                c = g * NBUF + b
                a = c + K
                ba = (b + K) % NBUF
                w = a - NBUF

                @pl.when(a < n_chunks)
                def _():
                    @pl.when(w >= 0)
                    def _():
                        write(w, ba).wait()
                    gather(a, ba).start()

                gather(c, b).wait()
                write(c, b).start()
            return carry

        lax.fori_loop(0, n_chunks // NBUF, step, 0)

        for c in range(n_chunks - NBUF, n_chunks):
            write(c, c % NBUF).wait()

    return k


def kernel(inputs, table):
    S0, S1 = inputs.shape
    B = S0 * S1
    V, D = table.shape
    W = 128
    chunk = 128
    n_chunks = B // (NW * chunk)
    idx = inputs.reshape(NW, n_chunks, chunk).astype(jnp.int32)
    table_p = jnp.pad(table, ((0, 0), (0, W - D)))
    out_p = _make(B, W, n_chunks, chunk)(idx, table_p)
    return out_p.reshape(S0, S1, W)[:, :, :D]
